# baseline (speedup 1.0000x reference)
import functools

import jax
import jax.numpy as jnp
from jax import lax
from jax.experimental import pallas as pl
from jax.experimental.pallas import tpu as pltpu
from jax.experimental.pallas import tpu_sc as plsc

NS = 16
NC = 2
NW = NC * NS
CH = 32
NBUF = 4
H = 2


def _mm_body(x_ref, w_ref, b_ref, o_ref, *, relu_in, relu_out):
    xv = x_ref[...]
    if relu_in:
        xv = jnp.maximum(xv, 0.0)
    y = jnp.dot(xv, w_ref[...], preferred_element_type=jnp.float32) + b_ref[...]
    if relu_out:
        y = jnp.maximum(y, 0.0)
    o_ref[...] = y


def _matmul_bias(x, W, b, relu_in, relu_out, blk):
    n, d_in = x.shape
    d_out = W.shape[1]
    grid = n // blk
    return pl.pallas_call(
        functools.partial(_mm_body, relu_in=relu_in, relu_out=relu_out),
        grid=(grid,),
        in_specs=[
            pl.BlockSpec((blk, d_in), lambda i: (i, 0)),
            pl.BlockSpec((d_in, d_out), lambda i: (0, 0)),
            pl.BlockSpec((1, d_out), lambda i: (0, 0)),
        ],
        out_specs=pl.BlockSpec((blk, d_out), lambda i: (i, 0)),
        out_shape=jax.ShapeDtypeStruct((n, d_out), jnp.float32),
    )(x, W, b.reshape(1, d_out))


def _p_specs(blk, d):
    return [
        pl.BlockSpec((1, blk, d), lambda i: (0, i, 0)),
        pl.BlockSpec((1, blk, d), lambda i: (1, i, 0)),
    ]


def _row(blk, d):
    return pl.BlockSpec((blk, d), lambda i: (i, 0))


def _dinv_body(p0_ref, p1_ref, o_ref):
    deg = p0_ref[0] + p1_ref[0]
    o_ref[...] = jnp.where(deg > 0.0, lax.rsqrt(deg), 0.0)


def _dinv(degP, n, d, blk):
    return pl.pallas_call(
        _dinv_body,
        grid=(n // blk,),
        in_specs=_p_specs(blk, d),
        out_specs=_row(blk, d),
        out_shape=jax.ShapeDtypeStruct((n, d), jnp.float32),
    )(degP, degP)


def _scale_body(a_ref, b_ref, o_ref):
    o_ref[...] = a_ref[...] * b_ref[...]


def _scale(a, b, blk):
    n, d = a.shape
    return pl.pallas_call(
        _scale_body,
        grid=(n // blk,),
        in_specs=[_row(blk, d)] * 2,
        out_specs=_row(blk, d),
        out_shape=jax.ShapeDtypeStruct((n, d), jnp.float32),
    )(a, b)


def _first_body(c_ref, p0_ref, p1_ref, dinv_ref, tx0_ref,
                tx1_ref, y_ref, out_ref):
    dinv = dinv_ref[...]
    a = dinv * (p0_ref[0] + p1_ref[0])
    tx0 = tx0_ref[...]
    tx1 = 0.5 * (tx0 + a)
    tx1_ref[...] = tx1
    y_ref[...] = dinv * tx1
    out_ref[...] = c_ref[0, 0] * tx0 + c_ref[0, 1] * tx1


def _first_step(coeffs, P, dinv, tx0, blk):
    n, d = tx0.shape
    sds = jax.ShapeDtypeStruct((n, d), jnp.float32)
    return pl.pallas_call(
        _first_body,
        grid=(n // blk,),
        in_specs=[pl.BlockSpec(memory_space=pltpu.SMEM)] + _p_specs(blk, d)
                 + [_row(blk, d)] * 2,
        out_specs=[_row(blk, d)] * 3,
        out_shape=[sds, sds, sds],
    )(coeffs, P, P, dinv, tx0)


def _step_body(c_ref, p0_ref, p1_ref, dinv_ref, tx1_ref, tx0_ref, oin_ref,
               tx2_ref, y_ref, onew_ref, *, k):
    dinv = dinv_ref[...]
    a = dinv * (p0_ref[0] + p1_ref[0])
    tx2 = tx1_ref[...] + a - tx0_ref[...]
    tx2_ref[...] = tx2
    y_ref[...] = dinv * tx2
    onew_ref[...] = oin_ref[...] + c_ref[0, k] * tx2


def _cheb_step(coeffs, P, dinv, tx1, tx0, out, k, blk):
    n, d = tx0.shape
    sds = jax.ShapeDtypeStruct((n, d), jnp.float32)
    return pl.pallas_call(
        functools.partial(_step_body, k=k),
        grid=(n // blk,),
        in_specs=[pl.BlockSpec(memory_space=pltpu.SMEM)] + _p_specs(blk, d)
                 + [_row(blk, d)] * 4,
        out_specs=[_row(blk, d)] * 3,
        out_shape=[sds, sds, sds],
    )(coeffs, P, P, dinv, tx1, tx0, out)


def _post_conv_body(o_ref, dinv_ref, h_ref, y_ref):
    h = jnp.maximum(o_ref[...], 0.0)
    h_ref[...] = h
    y_ref[...] = dinv_ref[...] * h


def _post_conv(out, dinv, blk):
    n, d = out.shape
    sds = jax.ShapeDtypeStruct((n, d), jnp.float32)
    return pl.pallas_call(
        _post_conv_body,
        grid=(n // blk,),
        in_specs=[_row(blk, d)] * 2,
        out_specs=[_row(blk, d)] * 2,
        out_shape=[sds, sds],
    )(out, dinv)


def _make_spmm(n, d, n_acc, nch):
    rpt = n_acc // NS
    outer = nch // NBUF

    def body(y_hbm, g_hbm, s_hbm, z_hbm, p_hbm, gv, sv, rows, acc, sg, ss):
        c = lax.axis_index("c")
        s = lax.axis_index("s")
        t = c * NS + s
        pltpu.sync_copy(g_hbm.at[t], gv)
        pltpu.sync_copy(s_hbm.at[t], sv)
        pltpu.sync_copy(z_hbm, acc.at[pl.ds(s * rpt, rpt)])
        plsc.subcore_barrier()

        def g_issue(j, b):
            pltpu.async_copy(y_hbm.at[gv.at[j]], rows.at[b], sg.at[b])

        def g_wait(j, b):
            pltpu.make_async_copy(y_hbm.at[gv.at[j]], rows.at[b],
                                  sg.at[b]).wait()

        def s_issue(j, b):
            pltpu.async_copy(rows.at[b], acc.at[sv.at[j]], ss.at[b], add=True)

        def s_wait(j, b):
            pltpu.make_async_copy(rows.at[b], acc.at[sv.at[j]],
                                  ss.at[b]).wait()

        for b in range(NBUF):
            g_issue(b, b)
            if b >= H:
                g_wait(b - H, b - H)
                s_issue(b - H, b - H)

        def grp(i, carry):
            for b in range(NBUF):
                j = i * NBUF + b
                s_wait(j - NBUF, b)
                g_issue(j, b)
                g_wait(j - H, (b - H) % NBUF)
                s_issue(j - H, (b - H) % NBUF)
            return carry

        lax.fori_loop(1, outer, grp, 0)

        for t2 in range(nch, nch + H):
            s_wait(t2 - NBUF, t2 % NBUF)
            g_wait(t2 - H, (t2 - H) % NBUF)
            s_issue(t2 - H, (t2 - H) % NBUF)
        for t2 in range(nch + H, nch + NBUF):
            s_wait(t2 - NBUF, t2 % NBUF)

        plsc.subcore_barrier()
        pltpu.sync_copy(acc.at[pl.ds(s * rpt, rpt)],
                        p_hbm.at[c, pl.ds(s * rpt, rpt)])

    return pl.kernel(
        body,
        out_type=jax.ShapeDtypeStruct((NC, n_acc, d), jnp.float32),
        mesh=plsc.VectorSubcoreMesh(core_axis_name="c", subcore_axis_name="s"),
        compiler_params=pltpu.CompilerParams(use_tc_tiling_on_sc=False),
        scratch_types=[
            pltpu.VMEM((nch, CH), jnp.int32),
            pltpu.VMEM((nch, CH), jnp.int32),
            pltpu.VMEM((NBUF, CH, d), jnp.float32),
            pltpu.VMEM_SHARED((n_acc, d), jnp.float32),
            pltpu.SemaphoreType.DMA((NBUF,)),
            pltpu.SemaphoreType.DMA((NBUF,)),
        ],
    )


def kernel(x, edge_index, W1, b1, W2, b2, coeffs1, coeffs2):
    n, d_in = x.shape
    e = edge_index.shape[1]
    k_max = coeffs1.shape[0] - 1
    d = W1.shape[1]
    blk = 2000 if n % 2000 == 0 else n

    epc = NW * CH
    e_pad = -(-e // (epc * NBUF)) * (epc * NBUF)
    nch = e_pad // epc
    pad = e_pad - e
    n_acc = -(-(n + 1) // (NS * 8)) * (NS * 8)

    src = edge_index[0]
    dst = edge_index[1]
    zeros_i = jnp.zeros((pad,), jnp.int32)
    trash_i = jnp.full((pad,), n, jnp.int32)
    gsrc = jnp.concatenate([src, zeros_i]).reshape(NW, nch, CH)
    ssrc = jnp.concatenate([src, trash_i]).reshape(NW, nch, CH)
    sdst = jnp.concatenate([dst, trash_i]).reshape(NW, nch, CH)

    zeros_hbm = jnp.zeros((n_acc // NS, d), jnp.float32)
    ones_y = jnp.ones((n, d), jnp.float32)

    spmm = _make_spmm(n, d, n_acc, nch)

    degP = spmm(ones_y, gsrc, ssrc, zeros_hbm)
    dinv = _dinv(degP, n, d, blk)

    h = _matmul_bias(x, W1, b1, relu_in=False, relu_out=True, blk=blk)

    tx0 = h
    y = _scale(dinv, h, blk)
    out = None
    for conv in range(2):
        coeffs = (coeffs1 if conv == 0 else coeffs2).reshape(1, k_max + 1)
        P = spmm(y, gsrc, sdst, zeros_hbm)
        tx1, y, out = _first_step(coeffs, P, dinv, tx0, blk)
        for k in range(2, k_max + 1):
            P = spmm(y, gsrc, sdst, zeros_hbm)
            tx2, y, out = _cheb_step(coeffs, P, dinv, tx1, tx0, out, k, blk)
            tx0, tx1 = tx1, tx2
        if conv == 0:
            tx0, y = _post_conv(out, dinv, blk)

    return _matmul_bias(out, W2, b2, relu_in=True, relu_out=False, blk=blk)


# speedup vs baseline: 1.2783x; 1.2783x over previous
import functools

import jax
import jax.numpy as jnp
from jax import lax
from jax.experimental import pallas as pl
from jax.experimental.pallas import tpu as pltpu
from jax.experimental.pallas import tpu_sc as plsc

NS = 16
NC = 2
CH = 32
NBUF = 4
H = 2


def _mm_body(x_ref, w_ref, b_ref, o_ref, *, relu_in, relu_out):
    xv = x_ref[...]
    if relu_in:
        xv = jnp.maximum(xv, 0.0)
    y = jnp.dot(xv, w_ref[...], preferred_element_type=jnp.float32) + b_ref[...]
    if relu_out:
        y = jnp.maximum(y, 0.0)
    o_ref[...] = y


def _matmul_bias(x, W, b, relu_in, relu_out, blk):
    n, d_in = x.shape
    d_out = W.shape[1]
    grid = n // blk
    return pl.pallas_call(
        functools.partial(_mm_body, relu_in=relu_in, relu_out=relu_out),
        grid=(grid,),
        in_specs=[
            pl.BlockSpec((blk, d_in), lambda i: (i, 0)),
            pl.BlockSpec((d_in, d_out), lambda i: (0, 0)),
            pl.BlockSpec((1, d_out), lambda i: (0, 0)),
        ],
        out_specs=pl.BlockSpec((blk, d_out), lambda i: (i, 0)),
        out_shape=jax.ShapeDtypeStruct((n, d_out), jnp.float32),
    )(x, W, b.reshape(1, d_out))


def _p_specs(blk, dh):
    return [
        pl.BlockSpec((1, blk, dh), lambda i: (0, i, 0)),
        pl.BlockSpec((1, blk, dh), lambda i: (1, i, 0)),
    ]


def _row(blk, d):
    return pl.BlockSpec((blk, d), lambda i: (i, 0))


def _ysplit(blk, dh):
    return pl.BlockSpec((NC, blk, dh), lambda i: (0, i, 0))


def _write_y(y_ref, v, dh):
    y_ref[0] = v[:, :dh]
    y_ref[1] = v[:, dh:]


def _dinv_body(p0_ref, o_ref):
    deg = p0_ref[0]
    r = jnp.where(deg > 0.0, lax.rsqrt(deg), 0.0)
    o_ref[...] = jnp.concatenate([r, r], axis=-1)


def _dinv(degP, n, d, blk):
    dh = d // 2
    return pl.pallas_call(
        _dinv_body,
        grid=(n // blk,),
        in_specs=_p_specs(blk, dh)[:1],
        out_specs=_row(blk, d),
        out_shape=jax.ShapeDtypeStruct((n, d), jnp.float32),
    )(degP)


def _make_y_body(dinv_ref, h_ref, y_ref, *, dh):
    _write_y(y_ref, dinv_ref[...] * h_ref[...], dh)


def _make_y(dinv, h, blk):
    n, d = h.shape
    dh = d // 2
    return pl.pallas_call(
        functools.partial(_make_y_body, dh=dh),
        grid=(n // blk,),
        in_specs=[_row(blk, d)] * 2,
        out_specs=_ysplit(blk, dh),
        out_shape=jax.ShapeDtypeStruct((NC, n, dh), jnp.float32),
    )(dinv, h)


def _first_body(c_ref, p0_ref, p1_ref, dinv_ref, tx0_ref,
                tx1_ref, y_ref, *, dh):
    dinv = dinv_ref[...]
    a = dinv * jnp.concatenate([p0_ref[0], p1_ref[0]], axis=-1)
    tx1 = 0.5 * (tx0_ref[...] + a)
    tx1_ref[...] = tx1
    _write_y(y_ref, dinv * tx1, dh)


def _first_step(coeffs, P, dinv, tx0, blk):
    n, d = tx0.shape
    dh = d // 2
    sds = jax.ShapeDtypeStruct((n, d), jnp.float32)
    ysds = jax.ShapeDtypeStruct((NC, n, dh), jnp.float32)
    return pl.pallas_call(
        functools.partial(_first_body, dh=dh),
        grid=(n // blk,),
        in_specs=[pl.BlockSpec(memory_space=pltpu.SMEM)] + _p_specs(blk, dh)
                 + [_row(blk, d)] * 2,
        out_specs=[_row(blk, d), _ysplit(blk, dh)],
        out_shape=[sds, ysds],
    )(coeffs, P, P, dinv, tx0)


def _out_init_body(c_ref, tx0_ref, tx1_ref, out_ref):
    out_ref[...] = c_ref[0, 0] * tx0_ref[...] + c_ref[0, 1] * tx1_ref[...]


def _out_init(coeffs, tx0, tx1, blk):
    n, d = tx0.shape
    return pl.pallas_call(
        _out_init_body,
        grid=(n // blk,),
        in_specs=[pl.BlockSpec(memory_space=pltpu.SMEM)] + [_row(blk, d)] * 2,
        out_specs=_row(blk, d),
        out_shape=jax.ShapeDtypeStruct((n, d), jnp.float32),
    )(coeffs, tx0, tx1)


def _step_body(c_ref, p0_ref, p1_ref, dinv_ref, tx1_ref, tx0_ref,
               tx2_ref, y_ref, *, dh):
    dinv = dinv_ref[...]
    a = dinv * jnp.concatenate([p0_ref[0], p1_ref[0]], axis=-1)
    tx2 = tx1_ref[...] + a - tx0_ref[...]
    tx2_ref[...] = tx2
    _write_y(y_ref, dinv * tx2, dh)


def _cheb_step(coeffs, P, dinv, tx1, tx0, blk):
    n, d = tx0.shape
    dh = d // 2
    sds = jax.ShapeDtypeStruct((n, d), jnp.float32)
    ysds = jax.ShapeDtypeStruct((NC, n, dh), jnp.float32)
    return pl.pallas_call(
        functools.partial(_step_body, dh=dh),
        grid=(n // blk,),
        in_specs=[pl.BlockSpec(memory_space=pltpu.SMEM)] + _p_specs(blk, dh)
                 + [_row(blk, d)] * 3,
        out_specs=[_row(blk, d), _ysplit(blk, dh)],
        out_shape=[sds, ysds],
    )(coeffs, P, P, dinv, tx1, tx0)


def _out_step_body(c_ref, oin_ref, tx2_ref, onew_ref, *, k):
    onew_ref[...] = oin_ref[...] + c_ref[0, k] * tx2_ref[...]


def _out_step(coeffs, out, tx2, k, blk):
    n, d = out.shape
    return pl.pallas_call(
        functools.partial(_out_step_body, k=k),
        grid=(n // blk,),
        in_specs=[pl.BlockSpec(memory_space=pltpu.SMEM)] + [_row(blk, d)] * 2,
        out_specs=_row(blk, d),
        out_shape=jax.ShapeDtypeStruct((n, d), jnp.float32),
    )(coeffs, out, tx2)


def _post_conv_body(o_ref, dinv_ref, h_ref, y_ref, *, dh):
    h = jnp.maximum(o_ref[...], 0.0)
    h_ref[...] = h
    _write_y(y_ref, dinv_ref[...] * h, dh)


def _post_conv(out, dinv, blk):
    n, d = out.shape
    dh = d // 2
    return pl.pallas_call(
        functools.partial(_post_conv_body, dh=dh),
        grid=(n // blk,),
        in_specs=[_row(blk, d)] * 2,
        out_specs=[_row(blk, d), _ysplit(blk, dh)],
        out_shape=[jax.ShapeDtypeStruct((n, d), jnp.float32),
                   jax.ShapeDtypeStruct((NC, n, dh), jnp.float32)],
    )(out, dinv)


def _make_spmm(n, dh, n_acc, nch):
    rpt = n_acc // NS
    outer = nch // NBUF

    def body(y_hbm, g_hbm, s_hbm, z_hbm, p_hbm, gv, sv, rows, acc, sg, ss):
        c = lax.axis_index("c")
        s = lax.axis_index("s")
        pltpu.sync_copy(g_hbm.at[s], gv)
        pltpu.sync_copy(s_hbm.at[s], sv)
        pltpu.sync_copy(z_hbm, acc.at[pl.ds(s * rpt, rpt)])
        plsc.subcore_barrier()

        yv = y_hbm.at[c]

        def g_issue(j, b):
            pltpu.async_copy(yv.at[gv.at[j]], rows.at[b], sg.at[b])

        def g_wait(j, b):
            pltpu.make_async_copy(yv.at[gv.at[j]], rows.at[b], sg.at[b]).wait()

        def s_issue(j, b):
            pltpu.async_copy(rows.at[b], acc.at[sv.at[j]], ss.at[b], add=True)

        def s_wait(j, b):
            pltpu.make_async_copy(rows.at[b], acc.at[sv.at[j]],
                                  ss.at[b]).wait()

        for b in range(NBUF):
            g_issue(b, b)
            if b >= H:
                g_wait(b - H, b - H)
                s_issue(b - H, b - H)

        def grp(i, carry):
            for b in range(NBUF):
                j = i * NBUF + b
                s_wait(j - NBUF, b)
                g_issue(j, b)
                g_wait(j - H, (b - H) % NBUF)
                s_issue(j - H, (b - H) % NBUF)
            return carry

        lax.fori_loop(1, outer, grp, 0)

        for t in range(nch, nch + H):
            s_wait(t - NBUF, t % NBUF)
            g_wait(t - H, (t - H) % NBUF)
            s_issue(t - H, (t - H) % NBUF)
        for t in range(nch + H, nch + NBUF):
            s_wait(t - NBUF, t % NBUF)

        plsc.subcore_barrier()
        pltpu.sync_copy(acc.at[pl.ds(s * rpt, rpt)],
                        p_hbm.at[c, pl.ds(s * rpt, rpt)])

    return pl.kernel(
        body,
        out_type=jax.ShapeDtypeStruct((NC, n_acc, dh), jnp.float32),
        mesh=plsc.VectorSubcoreMesh(core_axis_name="c", subcore_axis_name="s"),
        compiler_params=pltpu.CompilerParams(use_tc_tiling_on_sc=False),
        scratch_types=[
            pltpu.VMEM((nch, CH), jnp.int32),
            pltpu.VMEM((nch, CH), jnp.int32),
            pltpu.VMEM((NBUF, CH, dh), jnp.float32),
            pltpu.VMEM_SHARED((n_acc, dh), jnp.float32),
            pltpu.SemaphoreType.DMA((NBUF,)),
            pltpu.SemaphoreType.DMA((NBUF,)),
        ],
    )


def kernel(x, edge_index, W1, b1, W2, b2, coeffs1, coeffs2):
    n, d_in = x.shape
    e = edge_index.shape[1]
    k_max = coeffs1.shape[0] - 1
    d = W1.shape[1]
    dh = d // 2
    blk = 2000 if n % 2000 == 0 else n

    ept = NS * CH
    e_pad = -(-e // (ept * NBUF)) * (ept * NBUF)
    nch = e_pad // ept
    pad = e_pad - e
    n_acc = -(-(n + 1) // (NS * 8)) * (NS * 8)

    src = edge_index[0]
    dst = edge_index[1]
    zeros_i = jnp.zeros((pad,), jnp.int32)
    trash_i = jnp.full((pad,), n, jnp.int32)
    gsrc = jnp.concatenate([src, zeros_i]).reshape(NS, nch, CH)
    ssrc = jnp.concatenate([src, trash_i]).reshape(NS, nch, CH)
    sdst = jnp.concatenate([dst, trash_i]).reshape(NS, nch, CH)

    zeros_hbm = jnp.zeros((n_acc // NS, dh), jnp.float32)
    ones_y = jnp.ones((NC, n, dh), jnp.float32)

    spmm = _make_spmm(n, dh, n_acc, nch)

    degP = spmm(ones_y, gsrc, ssrc, zeros_hbm)
    dinv = _dinv(degP, n, d, blk)

    h = _matmul_bias(x, W1, b1, relu_in=False, relu_out=True, blk=blk)

    tx0 = h
    y = _make_y(dinv, h, blk)
    out = None
    for conv in range(2):
        coeffs = (coeffs1 if conv == 0 else coeffs2).reshape(1, k_max + 1)
        P = spmm(y, gsrc, sdst, zeros_hbm)
        tx1, y = _first_step(coeffs, P, dinv, tx0, blk)
        out = _out_init(coeffs, tx0, tx1, blk)
        for k in range(2, k_max + 1):
            P = spmm(y, gsrc, sdst, zeros_hbm)
            tx2, y = _cheb_step(coeffs, P, dinv, tx1, tx0, blk)
            out = _out_step(coeffs, out, tx2, k, blk)
            tx0, tx1 = tx1, tx2
        if conv == 0:
            tx0, y = _post_conv(out, dinv, blk)

    return _matmul_bias(out, W2, b2, relu_in=True, relu_out=False, blk=blk)


# speedup vs baseline: 1.6374x; 1.2809x over previous
import functools

import jax
import jax.numpy as jnp
from jax import lax
from jax.experimental import pallas as pl
from jax.experimental.pallas import tpu as pltpu
from jax.experimental.pallas import tpu_sc as plsc

NS = 16
NC = 2
CH = 40
NBUF = 4
H = 2


def _mm_body(x_ref, w_ref, b_ref, o_ref, *, relu_in, relu_out):
    xv = x_ref[...]
    if relu_in:
        xv = jnp.maximum(xv, 0.0)
    y = jnp.dot(xv, w_ref[...], preferred_element_type=jnp.float32) + b_ref[...]
    if relu_out:
        y = jnp.maximum(y, 0.0)
    o_ref[...] = y


def _matmul_bias(x, W, b, relu_in, relu_out, blk):
    n, d_in = x.shape
    d_out = W.shape[1]
    grid = n // blk
    return pl.pallas_call(
        functools.partial(_mm_body, relu_in=relu_in, relu_out=relu_out),
        grid=(grid,),
        in_specs=[
            pl.BlockSpec((blk, d_in), lambda i: (i, 0)),
            pl.BlockSpec((d_in, d_out), lambda i: (0, 0)),
            pl.BlockSpec((1, d_out), lambda i: (0, 0)),
        ],
        out_specs=pl.BlockSpec((blk, d_out), lambda i: (i, 0)),
        out_shape=jax.ShapeDtypeStruct((n, d_out), jnp.float32),
    )(x, W, b.reshape(1, d_out))


def _p_specs(blk, dh):
    return [
        pl.BlockSpec((1, blk, dh), lambda i: (0, i, 0)),
        pl.BlockSpec((1, blk, dh), lambda i: (1, i, 0)),
    ]


def _row(blk, d):
    return pl.BlockSpec((blk, d), lambda i: (i, 0))


def _ysplit(blk, dh):
    return pl.BlockSpec((NC, blk, dh), lambda i: (0, i, 0))


def _write_y(y_ref, v, dh):
    y_ref[0] = v[:, :dh]
    y_ref[1] = v[:, dh:]


def _dinv_body(p0_ref, o_ref):
    deg = p0_ref[0]
    r = jnp.where(deg > 0.0, lax.rsqrt(deg), 0.0)
    o_ref[...] = jnp.concatenate([r, r], axis=-1)


def _dinv(degP, n, d, blk):
    dh = d // 2
    return pl.pallas_call(
        _dinv_body,
        grid=(n // blk,),
        in_specs=_p_specs(blk, dh)[:1],
        out_specs=_row(blk, d),
        out_shape=jax.ShapeDtypeStruct((n, d), jnp.float32),
    )(degP)


def _make_y_body(dinv_ref, h_ref, y_ref, *, dh):
    _write_y(y_ref, dinv_ref[...] * h_ref[...], dh)


def _make_y(dinv, h, blk):
    n, d = h.shape
    dh = d // 2
    return pl.pallas_call(
        functools.partial(_make_y_body, dh=dh),
        grid=(n // blk,),
        in_specs=[_row(blk, d)] * 2,
        out_specs=_ysplit(blk, dh),
        out_shape=jax.ShapeDtypeStruct((NC, n, dh), jnp.float32),
    )(dinv, h)


def _first_body(c_ref, p0_ref, p1_ref, dinv_ref, tx0_ref,
                tx1_ref, y_ref, out_ref, *, dh):
    dinv = dinv_ref[...]
    a = dinv * jnp.concatenate([p0_ref[0], p1_ref[0]], axis=-1)
    tx0 = tx0_ref[...]
    tx1 = 0.5 * (tx0 + a)
    tx1_ref[...] = tx1
    _write_y(y_ref, dinv * tx1, dh)
    out_ref[...] = c_ref[0, 0] * tx0 + c_ref[0, 1] * tx1


def _first_step(coeffs, P, dinv, tx0, blk):
    n, d = tx0.shape
    dh = d // 2
    sds = jax.ShapeDtypeStruct((n, d), jnp.float32)
    ysds = jax.ShapeDtypeStruct((NC, n, dh), jnp.float32)
    return pl.pallas_call(
        functools.partial(_first_body, dh=dh),
        grid=(n // blk,),
        in_specs=[pl.BlockSpec(memory_space=pltpu.SMEM)] + _p_specs(blk, dh)
                 + [_row(blk, d)] * 2,
        out_specs=[_row(blk, d), _ysplit(blk, dh), _row(blk, d)],
        out_shape=[sds, ysds, sds],
    )(coeffs, P, P, dinv, tx0)


def _step_body(c_ref, p0_ref, p1_ref, dinv_ref, tx1_ref, tx0_ref, oin_ref,
               tx2_ref, y_ref, onew_ref, *, k, dh):
    dinv = dinv_ref[...]
    a = dinv * jnp.concatenate([p0_ref[0], p1_ref[0]], axis=-1)
    tx2 = tx1_ref[...] + a - tx0_ref[...]
    tx2_ref[...] = tx2
    _write_y(y_ref, dinv * tx2, dh)
    onew_ref[...] = oin_ref[...] + c_ref[0, k] * tx2


def _cheb_step(coeffs, P, dinv, tx1, tx0, out, k, blk):
    n, d = tx0.shape
    dh = d // 2
    sds = jax.ShapeDtypeStruct((n, d), jnp.float32)
    ysds = jax.ShapeDtypeStruct((NC, n, dh), jnp.float32)
    return pl.pallas_call(
        functools.partial(_step_body, k=k, dh=dh),
        grid=(n // blk,),
        in_specs=[pl.BlockSpec(memory_space=pltpu.SMEM)] + _p_specs(blk, dh)
                 + [_row(blk, d)] * 4,
        out_specs=[_row(blk, d), _ysplit(blk, dh), _row(blk, d)],
        out_shape=[sds, ysds, sds],
    )(coeffs, P, P, dinv, tx1, tx0, out)


def _post_conv_body(o_ref, dinv_ref, h_ref, y_ref, *, dh):
    h = jnp.maximum(o_ref[...], 0.0)
    h_ref[...] = h
    _write_y(y_ref, dinv_ref[...] * h, dh)


def _post_conv(out, dinv, blk):
    n, d = out.shape
    dh = d // 2
    return pl.pallas_call(
        functools.partial(_post_conv_body, dh=dh),
        grid=(n // blk,),
        in_specs=[_row(blk, d)] * 2,
        out_specs=[_row(blk, d), _ysplit(blk, dh)],
        out_shape=[jax.ShapeDtypeStruct((n, d), jnp.float32),
                   jax.ShapeDtypeStruct((NC, n, dh), jnp.float32)],
    )(out, dinv)


def _make_spmm(n, dh, n_acc, nch):
    rpt = n_acc // NS
    outer = nch // NBUF

    def body(y_hbm, g_hbm, s_hbm, z_hbm, p_hbm, gv, sv, rows, acc, sg, ss):
        c = lax.axis_index("c")
        s = lax.axis_index("s")
        pltpu.sync_copy(g_hbm.at[s], gv)
        pltpu.sync_copy(s_hbm.at[s], sv)
        pltpu.sync_copy(z_hbm, acc.at[pl.ds(s * rpt, rpt)])
        plsc.subcore_barrier()

        yv = y_hbm.at[c]

        def g_issue(j, b):
            pltpu.async_copy(yv.at[gv.at[j]], rows.at[b], sg.at[b])

        def g_wait(j, b):
            pltpu.make_async_copy(yv.at[gv.at[j]], rows.at[b], sg.at[b]).wait()

        def s_issue(j, b):
            pltpu.async_copy(rows.at[b], acc.at[sv.at[j]], ss.at[b], add=True)

        def s_wait(j, b):
            pltpu.make_async_copy(rows.at[b], acc.at[sv.at[j]],
                                  ss.at[b]).wait()

        for b in range(NBUF):
            g_issue(b, b)
            if b >= H:
                g_wait(b - H, b - H)
                s_issue(b - H, b - H)

        def grp(i, carry):
            for b in range(NBUF):
                j = i * NBUF + b
                s_wait(j - NBUF, b)
                g_issue(j, b)
                g_wait(j - H, (b - H) % NBUF)
                s_issue(j - H, (b - H) % NBUF)
            return carry

        lax.fori_loop(1, outer, grp, 0)

        for t in range(nch, nch + H):
            s_wait(t - NBUF, t % NBUF)
            g_wait(t - H, (t - H) % NBUF)
            s_issue(t - H, (t - H) % NBUF)
        for t in range(nch + H, nch + NBUF):
            s_wait(t - NBUF, t % NBUF)

        plsc.subcore_barrier()
        pltpu.sync_copy(acc.at[pl.ds(s * rpt, rpt)],
                        p_hbm.at[c, pl.ds(s * rpt, rpt)])

    return pl.kernel(
        body,
        out_type=jax.ShapeDtypeStruct((NC, n_acc, dh), jnp.float32),
        mesh=plsc.VectorSubcoreMesh(core_axis_name="c", subcore_axis_name="s"),
        compiler_params=pltpu.CompilerParams(use_tc_tiling_on_sc=False),
        scratch_types=[
            pltpu.VMEM((nch, CH), jnp.int32),
            pltpu.VMEM((nch, CH), jnp.int32),
            pltpu.VMEM((NBUF, CH, dh), jnp.float32),
            pltpu.VMEM_SHARED((n_acc, dh), jnp.float32),
            pltpu.SemaphoreType.DMA((NBUF,)),
            pltpu.SemaphoreType.DMA((NBUF,)),
        ],
    )


def kernel(x, edge_index, W1, b1, W2, b2, coeffs1, coeffs2):
    n, d_in = x.shape
    e = edge_index.shape[1]
    k_max = coeffs1.shape[0] - 1
    d = W1.shape[1]
    dh = d // 2
    blk = 2000 if n % 2000 == 0 else n

    ept = NS * CH
    e_pad = -(-e // (ept * NBUF)) * (ept * NBUF)
    nch = e_pad // ept
    pad = e_pad - e
    n_acc = -(-(n + 1) // (NS * 8)) * (NS * 8)

    src = edge_index[0]
    dst = edge_index[1]
    zeros_i = jnp.zeros((pad,), jnp.int32)
    trash_i = jnp.full((pad,), n, jnp.int32)
    gsrc = jnp.concatenate([src, zeros_i]).reshape(NS, nch, CH)
    ssrc = jnp.concatenate([src, trash_i]).reshape(NS, nch, CH)
    sdst = jnp.concatenate([dst, trash_i]).reshape(NS, nch, CH)

    zeros_hbm = jnp.zeros((n_acc // NS, dh), jnp.float32)
    ones_y = jnp.ones((NC, n, dh), jnp.float32)

    spmm = _make_spmm(n, dh, n_acc, nch)

    degP = spmm(ones_y, gsrc, ssrc, zeros_hbm)
    dinv = _dinv(degP, n, d, blk)

    h = _matmul_bias(x, W1, b1, relu_in=False, relu_out=True, blk=blk)

    tx0 = h
    y = _make_y(dinv, h, blk)
    out = None
    for conv in range(2):
        coeffs = (coeffs1 if conv == 0 else coeffs2).reshape(1, k_max + 1)
        P = spmm(y, gsrc, sdst, zeros_hbm)
        tx1, y, out = _first_step(coeffs, P, dinv, tx0, blk)
        for k in range(2, k_max + 1):
            P = spmm(y, gsrc, sdst, zeros_hbm)
            tx2, y, out = _cheb_step(coeffs, P, dinv, tx1, tx0, out, k, blk)
            tx0, tx1 = tx1, tx2
        if conv == 0:
            tx0, y = _post_conv(out, dinv, blk)

    return _matmul_bias(out, W2, b2, relu_in=True, relu_out=False, blk=blk)
